# baseline (speedup 1.0000x reference)
import functools

import jax
import jax.numpy as jnp
from jax import lax
from jax.experimental import pallas as pl
from jax.experimental.pallas import tpu as pltpu
from jax.experimental.pallas import tpu_sc as plsc

_ROWS = (2, 4, 5)


def _gather_body(x_hbm, out_hbm, row_v):
    wid = lax.axis_index("s") * 2 + lax.axis_index("c")
    for w, src in enumerate(_ROWS):
        @pl.when(wid == w)
        def _():
            pltpu.sync_copy(x_hbm.at[pl.ds(src, 1)], row_v)
            pltpu.sync_copy(row_v, out_hbm.at[pl.ds(w, 1)])


@jax.jit
def kernel(x):
    mesh = plsc.VectorSubcoreMesh(core_axis_name="c", subcore_axis_name="s")
    run = pl.kernel(
        _gather_body,
        mesh=mesh,
        out_type=jax.ShapeDtypeStruct((3, 128), jnp.float32),
        scratch_types=[pltpu.VMEM((1, 128), jnp.float32)],
    )
    return run(x)


# speedup vs baseline: 1.0341x; 1.0341x over previous
import jax
import jax.numpy as jnp
from jax import lax
from jax.experimental import pallas as pl
from jax.experimental.pallas import tpu as pltpu
from jax.experimental.pallas import tpu_sc as plsc


def _gather_body(x_hbm, out_hbm):
    cid = lax.axis_index("c")

    @pl.when(cid == 0)
    def _():
        pltpu.sync_copy(x_hbm.at[pl.ds(2, 1)], out_hbm.at[pl.ds(0, 1)])
        pltpu.sync_copy(x_hbm.at[pl.ds(4, 2)], out_hbm.at[pl.ds(1, 2)])


@jax.jit
def kernel(x):
    mesh = plsc.ScalarSubcoreMesh(axis_name="c", num_cores=2)
    run = pl.kernel(
        _gather_body,
        mesh=mesh,
        out_type=jax.ShapeDtypeStruct((3, 128), jnp.float32),
    )
    return run(x)


# speedup vs baseline: 1.0864x; 1.0506x over previous
import jax
import jax.numpy as jnp
from jax import lax
from jax.experimental import pallas as pl
from jax.experimental.pallas import tpu as pltpu
from jax.experimental.pallas import tpu_sc as plsc


def _gather_body(x_hbm, out_hbm):
    cid = lax.axis_index("c")

    @pl.when(cid == 0)
    def _():
        pltpu.sync_copy(x_hbm.at[pl.ds(2, 1)], out_hbm.at[pl.ds(0, 1)])

    @pl.when(cid == 1)
    def _():
        pltpu.sync_copy(x_hbm.at[pl.ds(4, 2)], out_hbm.at[pl.ds(1, 2)])


@jax.jit
def kernel(x):
    mesh = plsc.ScalarSubcoreMesh(axis_name="c", num_cores=2)
    run = pl.kernel(
        _gather_body,
        mesh=mesh,
        out_type=jax.ShapeDtypeStruct((3, 128), jnp.float32),
    )
    return run(x)


# speedup vs baseline: 1.1762x; 1.0827x over previous
import jax
import jax.numpy as jnp
from jax import lax
from jax.experimental import pallas as pl
from jax.experimental.pallas import tpu as pltpu
from jax.experimental.pallas import tpu_sc as plsc


def _gather_body(x_hbm, out_hbm, sem0, sem1):
    c0 = pltpu.make_async_copy(
        x_hbm.at[pl.ds(2, 1)], out_hbm.at[pl.ds(0, 1)], sem0
    )
    c1 = pltpu.make_async_copy(
        x_hbm.at[pl.ds(4, 2)], out_hbm.at[pl.ds(1, 2)], sem1
    )
    c0.start()
    c1.start()
    c0.wait()
    c1.wait()


@jax.jit
def kernel(x):
    mesh = plsc.ScalarSubcoreMesh(axis_name="c", num_cores=1)
    run = pl.kernel(
        _gather_body,
        mesh=mesh,
        out_type=jax.ShapeDtypeStruct((3, 128), jnp.float32),
        scratch_types=[pltpu.SemaphoreType.DMA, pltpu.SemaphoreType.DMA],
    )
    return run(x)
